# baseline (speedup 1.0000x reference)
import functools

import jax
import jax.numpy as jnp
from jax import lax
from jax.experimental import pallas as pl
from jax.experimental.pallas import tpu as pltpu
from jax.experimental.pallas import tpu_sc as plsc

EMBED_D = 316
BATCH = 4096
HIST = 20
B_TOTAL = BATCH * HIST
GRAN = 32
GPR = 11
NUM_CORES = 2
NUM_SUBCORES = 16
NW = NUM_CORES * NUM_SUBCORES
B_PER_W = B_TOTAL // NW
BCHUNK = 4
CHUNK = BCHUNK * HIST
N_CHUNKS = B_PER_W // CHUNK
NIDX = CHUNK * GPR
SLOT = GPR * GRAN

_mesh = plsc.VectorSubcoreMesh(core_axis_name="c", subcore_axis_name="s")


@functools.partial(
    pl.kernel,
    mesh=_mesh,
    out_type=jax.ShapeDtypeStruct((BATCH, HIST, EMBED_D), jnp.float32),
    scratch_types=[
        pltpu.VMEM((B_PER_W,), jnp.int32),
        pltpu.VMEM((CHUNK,), jnp.int32),
        pltpu.VMEM((CHUNK,), jnp.int32),
        pltpu.VMEM((NIDX,), jnp.int32),
        pltpu.VMEM((NIDX,), jnp.int32),
        pltpu.VMEM((NIDX, GRAN), jnp.float32),
        pltpu.VMEM((NIDX, GRAN), jnp.float32),
        pltpu.VMEM((BCHUNK, HIST, EMBED_D), jnp.float32),
        pltpu.VMEM((BCHUNK, HIST, EMBED_D), jnp.float32),
        pltpu.SemaphoreType.DMA,
        pltpu.SemaphoreType.DMA,
        pltpu.SemaphoreType.DMA,
        pltpu.SemaphoreType.DMA,
    ],
    compiler_params=pltpu.CompilerParams(
        use_tc_tiling_on_sc=False, needs_layout_passes=False
    ),
)
def _gather_kernel(idx_hbm, tview_hbm, out_hbm, idx_v, r0_v, r1_v,
                   gidx0_v, gidx1_v, fetch0_v, fetch1_v,
                   packed0_v, packed1_v, sem0, sem1, wsem0, wsem1):
    wid = lax.axis_index("s") * NUM_CORES + lax.axis_index("c")
    base = wid * B_PER_W
    bbase = wid * (B_PER_W // HIST)
    rs = (r0_v, r1_v)
    gidxs = (gidx0_v, gidx1_v)
    fetches = (fetch0_v, fetch1_v)
    packs = (packed0_v, packed1_v)
    sems = (sem0, sem1)
    wsems = (wsem0, wsem1)
    lanes = lax.iota(jnp.int32, 16)

    pltpu.sync_copy(idx_hbm.at[pl.ds(base, B_PER_W)], idx_v)

    def out_window(c):
        return out_hbm.at[pl.ds(bbase + c * BCHUNK, BCHUNK)]

    def issue(c, slot):
        gidx = gidxs[slot]
        for g in range(CHUNK // 16):
            a = idx_v[pl.ds(c * CHUNK + g * 16, 16)]
            u = a * EMBED_D
            q = u >> 5
            rs[slot][pl.ds(g * 16, 16)] = u & 31
            pos = (g * 16 + lanes) * GPR
            for j in range(GPR):
                plsc.store_scatter(gidx, [pos + j], q + j)
        pltpu.async_copy(tview_hbm.at[gidx], fetches[slot], sems[slot])

    def process(c, slot):
        fetch = fetches[slot]
        packed = packs[slot]
        rv = rs[slot]
        pltpu.make_async_copy(tview_hbm.at[gidxs[slot]], fetch,
                              sems[slot]).wait()

        @pl.when(c >= 2)
        def _():
            pltpu.make_async_copy(packed, out_window(c), wsems[slot]).wait()

        for bl in range(BCHUNK):
            def row(h, carry):
                rli = bl * HIST + h
                rj = plsc.load_gather(rv, [lanes * 0 + rli])
                srcb = rj + rli * SLOT
                for k in range(EMBED_D // 16 + 1):
                    off = min(k * 16, EMBED_D - 16)
                    t = srcb + (off + lanes)
                    v = plsc.load_gather(fetch, [t >> 5, t & 31])
                    packed[bl, h, pl.ds(off, 16)] = v
                return carry

            lax.fori_loop(0, HIST, row, 0)
        pltpu.async_copy(packed, out_window(c), wsems[slot])

    issue(0, 0)

    def pair(g, carry):
        c = 2 * g

        @pl.when(c + 1 < N_CHUNKS)
        def _():
            issue(c + 1, 1)

        process(c, 0)

        @pl.when(c + 2 < N_CHUNKS)
        def _():
            issue(c + 2, 0)

        @pl.when(c + 1 < N_CHUNKS)
        def _():
            process(c + 1, 1)

        return carry

    lax.fori_loop(0, N_CHUNKS // 2, pair, 0)

    pltpu.make_async_copy(packed0_v, out_window(N_CHUNKS - 2), wsem0).wait()
    pltpu.make_async_copy(packed1_v, out_window(N_CHUNKS - 1), wsem1).wait()


def kernel(indices, table):
    flat_idx = indices.reshape(-1)
    tview = table.reshape(-1, GRAN)
    return _gather_kernel(flat_idx, tview)


# speedup vs baseline: 2.5670x; 2.5670x over previous
import functools

import jax
import jax.numpy as jnp
from jax import lax
from jax.experimental import pallas as pl
from jax.experimental.pallas import tpu as pltpu
from jax.experimental.pallas import tpu_sc as plsc

EMBED_D = 316
BATCH = 4096
HIST = 20
B_TOTAL = BATCH * HIST
NUM_CORES = 2
NUM_SUBCORES = 16
NW = NUM_CORES * NUM_SUBCORES
B_PER_W = B_TOTAL // NW
CHUNK = 40
N_CHUNKS = B_PER_W // CHUNK

_mesh = plsc.VectorSubcoreMesh(core_axis_name="c", subcore_axis_name="s")


@functools.partial(
    pl.kernel,
    mesh=_mesh,
    out_type=jax.ShapeDtypeStruct((B_TOTAL, EMBED_D), jnp.float32),
    scratch_types=[
        pltpu.VMEM((B_PER_W,), jnp.int32),
        pltpu.VMEM((CHUNK, EMBED_D), jnp.float32),
        pltpu.VMEM((CHUNK, EMBED_D), jnp.float32),
        pltpu.SemaphoreType.DMA,
        pltpu.SemaphoreType.DMA,
        pltpu.SemaphoreType.DMA,
        pltpu.SemaphoreType.DMA,
    ],
    compiler_params=pltpu.CompilerParams(
        use_tc_tiling_on_sc=True, needs_layout_passes=False
    ),
)
def _gather_kernel(idx_hbm, table_hbm, out_hbm, idx_v,
                   packed0_v, packed1_v, sem0, sem1, wsem0, wsem1):
    wid = lax.axis_index("s") * NUM_CORES + lax.axis_index("c")
    base = wid * B_PER_W
    packs = (packed0_v, packed1_v)
    sems = (sem0, sem1)
    wsems = (wsem0, wsem1)
    lanes = lax.iota(jnp.int32, 16)

    pltpu.sync_copy(idx_hbm.at[pl.ds(base, B_PER_W)], idx_v)

    def out_window(c):
        return out_hbm.at[pl.ds(base + c * CHUNK, CHUNK)]

    def issue(c, slot):
        packed = packs[slot]
        sem = sems[slot]

        def row(r, carry):
            a = jnp.max(plsc.load_gather(idx_v, [lanes * 0 + c * CHUNK + r]))
            pltpu.async_copy(table_hbm.at[pl.ds(a, 1)],
                             packed.at[pl.ds(r, 1)], sem)
            return carry

        lax.fori_loop(0, CHUNK, row, 0)

    def process(c, slot):
        packed = packs[slot]
        pltpu.make_async_copy(table_hbm.at[pl.ds(0, CHUNK)], packed,
                              sems[slot]).wait()

        @pl.when(c >= 2)
        def _():
            pltpu.make_async_copy(packed, out_window(c), wsems[slot]).wait()

        pltpu.async_copy(packed, out_window(c), wsems[slot])

    issue(0, 0)

    def pair(g, carry):
        c = 2 * g

        @pl.when(c + 1 < N_CHUNKS)
        def _():
            issue(c + 1, 1)

        process(c, 0)

        @pl.when(c + 2 < N_CHUNKS)
        def _():
            issue(c + 2, 0)

        @pl.when(c + 1 < N_CHUNKS)
        def _():
            process(c + 1, 1)

        return carry

    lax.fori_loop(0, N_CHUNKS // 2, pair, 0)

    pltpu.make_async_copy(packed0_v, out_window(N_CHUNKS - 2), wsem0).wait()
    pltpu.make_async_copy(packed1_v, out_window(N_CHUNKS - 1), wsem1).wait()


def kernel(indices, table):
    flat_idx = indices.reshape(-1)
    out = _gather_kernel(flat_idx, table)
    return out.reshape(indices.shape + (table.shape[1],))
